# Initial kernel scaffold; baseline (speedup 1.0000x reference)
import jax
import jax.numpy as jnp
from jax.experimental import pallas as pl


def kernel(X, he, W1, b1, W2, b2):
    raise NotImplementedError("write your pallas kernel here")


# speedup vs baseline: 1.5265x; 1.5265x over previous
import functools

import jax
import jax.numpy as jnp
from jax import lax
from jax.experimental import pallas as pl
from jax.experimental.pallas import tpu as pltpu
from jax.experimental.pallas import tpu_sc as plsc

NC = 2
NS = 16
NW = NC * NS
LANES = 16


def _mesh():
    return plsc.VectorSubcoreMesh(core_axis_name="c", subcore_axis_name="s")


def _round_up(a, b):
    return (a + b - 1) // b * b


def _build_edges(X, he_flat, he_cols, M, Mp, K, Npad):
    N, D = X.shape
    P = K * (K - 1) // 2
    pairs = [(a, b) for a in range(K) for b in range(a + 1, K)]
    NJ = D // LANES
    CH = Mp // NW
    NB = CH // LANES
    SLAB = Npad + 2048
    SH = SLAB // NS
    WH = Npad // NS

    @functools.partial(
        pl.kernel,
        out_type=[
            jax.ShapeDtypeStruct((Mp,), jnp.int32),
            jax.ShapeDtypeStruct((Mp,), jnp.int32),
            jax.ShapeDtypeStruct((NC * Npad,), jnp.float32),
        ],
        mesh=_mesh(),
        scratch_types=[
            pltpu.VMEM((LANES * K,), jnp.int32),
            pltpu.VMEM((K * CH,), jnp.int32),
            pltpu.VMEM((LANES * K, D), jnp.float32),
            pltpu.VMEM((CH,), jnp.int32),
            pltpu.VMEM((CH,), jnp.int32),
            pltpu.VMEM((LANES,), jnp.float32),
            pltpu.VMEM((SH // 8,), jnp.float32),
            pltpu.VMEM_SHARED((SLAB,), jnp.float32),
            pltpu.SemaphoreType.DMA,
        ],
    )
    def k1(x_hbm, he_hbm, hcol_hbm, u_hbm, v_hbm, degw_hbm, he_v, hcol_v,
           rows_v, u_v, v_v, ones_v, zbuf, slab, sem):
        cid = lax.axis_index("c")
        sid = lax.axis_index("s")
        wid = sid * NC + cid
        base = wid * CH
        lanev = lax.iota(jnp.int32, LANES)
        zero16 = jnp.zeros((LANES,), jnp.float32)

        ones_v[pl.ds(0, LANES)] = jnp.ones((LANES,), jnp.float32)

        def zb_body(i, c):
            zbuf[pl.ds(i * LANES, LANES)] = zero16
            return c

        lax.fori_loop(0, SH // 8 // LANES, zb_body, 0)
        for k in range(K):
            pltpu.sync_copy(hcol_hbm.at[pl.ds(k * Mp + base, CH)],
                            hcol_v.at[pl.ds(k * CH, CH)])

        def batch(b, c):
            hbase = base + b * LANES
            pltpu.sync_copy(he_hbm.at[pl.ds(hbase * K, LANES * K)], he_v)
            pltpu.async_copy(x_hbm.at[he_v], rows_v, sem).wait()

            def eloop(e, besti):
                r0 = e * K
                ch = [[rows_v[r0 + k, pl.ds(j * LANES, LANES)]
                       for j in range(NJ)] for k in range(K)]
                s = []
                for (a, b2) in pairs:
                    acc = zero16
                    for j in range(NJ):
                        df = ch[a][j] - ch[b2][j]
                        acc = acc + df * df
                    for sh in (8, 4, 2, 1):
                        idx = jnp.bitwise_xor(lanev, jnp.int32(sh))
                        acc = acc + acc.at[idx].get(mode="promise_in_bounds")
                    s.append(acc)
                bv = s[0]
                bp = jnp.zeros((LANES,), jnp.int32)
                for p in range(1, P):
                    m = s[p] > bv
                    bv = jnp.where(m, s[p], bv)
                    bp = jnp.where(m, jnp.int32(p), bp)
                return jnp.where(lanev == e, bp, besti)

            besti = lax.fori_loop(0, LANES, eloop,
                                  jnp.zeros((LANES,), jnp.int32))

            hk = [hcol_v[pl.ds(k * CH + b * LANES, LANES)] for k in range(K)]
            u16 = hk[pairs[0][0]]
            v16 = hk[pairs[0][1]]
            for p in range(1, P):
                m = besti == p
                u16 = jnp.where(m, hk[pairs[p][0]], u16)
                v16 = jnp.where(m, hk[pairs[p][1]], v16)

            u_v[pl.ds(b * LANES, LANES)] = u16
            v_v[pl.ds(b * LANES, LANES)] = v16
            return c

        lax.fori_loop(0, NB, batch, 0)
        pltpu.sync_copy(u_v, u_hbm.at[pl.ds(base, CH)])
        pltpu.sync_copy(v_v, v_hbm.at[pl.ds(base, CH)])

        for z in range(8):
            pltpu.sync_copy(zbuf, slab.at[pl.ds(sid * SH + z * (SH // 8),
                                                SH // 8)])
        plsc.subcore_barrier()

        def dacc(t, c):
            ebase = base + t * LANES
            valid = (ebase + lanev) < M
            for idx_v in (u_v, v_v):
                idx = idx_v[pl.ds(t * LANES, LANES)]
                didx = jnp.where(valid, idx, jnp.int32(Npad))
                pltpu.sync_copy(ones_v, slab.at[didx], add=True)
            return c

        lax.fori_loop(0, NB, dacc, 0)
        plsc.subcore_barrier()
        pltpu.sync_copy(slab.at[pl.ds(sid * WH, WH)],
                        degw_hbm.at[pl.ds(cid * Npad + sid * WH, WH)])

    return k1(X, he_flat, he_cols)


def _scatter_pass(yflat, sidx_exp, didx_exp, Q, N, Npad):
    QH = Q // NC
    EpL = sidx_exp.shape[0]
    EL = EpL // NS
    GB = 2048
    NCH = EL // GB
    N16 = N * LANES
    NpadE = Npad * LANES
    SH = NpadE // NS
    ZB = 1024

    @functools.partial(
        pl.kernel,
        out_type=jax.ShapeDtypeStruct((Q * NpadE,), jnp.float32),
        mesh=_mesh(),
        scratch_types=[
            pltpu.VMEM((GB,), jnp.int32),
            pltpu.VMEM((GB,), jnp.int32),
            pltpu.VMEM((GB,), jnp.float32),
            pltpu.VMEM((ZB,), jnp.float32),
            pltpu.VMEM_SHARED((NpadE,), jnp.float32),
            pltpu.SemaphoreType.DMA,
        ],
    )
    def ks(y_hbm, src_hbm, dst_hbm, out_hbm, sidx_v, didx_v, gbuf, zbuf,
           slab, sem):
        cid = lax.axis_index("c")
        sid = lax.axis_index("s")
        ebase = sid * EL
        zerov = jnp.zeros((LANES,), jnp.float32)

        def zb_body(i, c):
            zbuf[pl.ds(i * LANES, LANES)] = zerov
            return c

        lax.fori_loop(0, ZB // LANES, zb_body, 0)

        def pass_body(p, c):
            q = cid * QH + p
            ybase = q * N16

            def zrow(r, c2):
                pltpu.sync_copy(zbuf, slab.at[pl.ds(sid * SH + r * ZB, ZB)])
                return c2

            lax.fori_loop(0, SH // ZB, zrow, 0)
            plsc.subcore_barrier()

            def chunk(j, c2):
                e0 = ebase + j * GB
                pltpu.sync_copy(src_hbm.at[pl.ds(e0, GB)], sidx_v)
                pltpu.sync_copy(dst_hbm.at[pl.ds(e0, GB)], didx_v)
                pltpu.sync_copy(y_hbm.at[pl.ds(ybase, N16)].at[sidx_v], gbuf)
                pltpu.sync_copy(gbuf, slab.at[didx_v], add=True)
                return c2

            lax.fori_loop(0, NCH, chunk, 0)
            plsc.subcore_barrier()

            s0 = sid * SH
            pltpu.sync_copy(slab.at[pl.ds(s0, SH)],
                            out_hbm.at[pl.ds(q * NpadE + s0, SH)])
            plsc.subcore_barrier()
            return c

        lax.fori_loop(0, QH, pass_body, 0)

    return ks(yflat, sidx_exp, didx_exp)


def _deg_finalize(degw2, Npad):
    R = Npad // 128
    BR = R // 7 if R % 7 == 0 else R
    G = R // BR

    def body(degw_ref, dis_ref, dinv_ref):
        deg = jnp.sum(degw_ref[...], axis=0) + 1.0
        dis_ref[...] = lax.rsqrt(deg)
        dinv_ref[...] = 1.0 / deg

    return pl.pallas_call(
        body,
        grid=(G,),
        in_specs=[pl.BlockSpec((NC, BR, 128), lambda i: (0, i, 0))],
        out_specs=[
            pl.BlockSpec((BR, 128), lambda i: (i, 0)),
            pl.BlockSpec((BR, 128), lambda i: (i, 0)),
        ],
        out_shape=[
            jax.ShapeDtypeStruct((R, 128), jnp.float32),
            jax.ShapeDtypeStruct((R, 128), jnp.float32),
        ],
    )(degw2)


def _dense1(X, W1, b1, disN):
    N, D = X.shape
    H = W1.shape[1]
    BR = 2000
    G = N // BR

    def body(x_ref, w_ref, b_ref, dis_ref, p_ref, y_ref):
        p = jnp.dot(x_ref[...], w_ref[...],
                    preferred_element_type=jnp.float32) + b_ref[...]
        p_ref[...] = p
        y_ref[...] = p * dis_ref[...][:, 0:1]

    return pl.pallas_call(
        body,
        grid=(G,),
        in_specs=[
            pl.BlockSpec((BR, D), lambda i: (i, 0)),
            pl.BlockSpec((D, H), lambda i: (0, 0)),
            pl.BlockSpec((1, H), lambda i: (0, 0)),
            pl.BlockSpec((BR, LANES), lambda i: (i, 0)),
        ],
        out_specs=[
            pl.BlockSpec((BR, H), lambda i: (i, 0)),
            pl.BlockSpec((BR, H), lambda i: (i, 0)),
        ],
        out_shape=[
            jax.ShapeDtypeStruct((N, H), jnp.float32),
            jax.ShapeDtypeStruct((N, H), jnp.float32),
        ],
    )(X, W1, b1, disN)


def _dense2(S1, P1, disN, dinvN, W2, b2):
    N, H = S1.shape
    C = W2.shape[1]
    BR = 2000
    G = N // BR

    def body(s_ref, p1_ref, dis_ref, dinv_ref, w_ref, b_ref, p2_ref, y_ref):
        dis = dis_ref[...][:, 0:1]
        dinv = dinv_ref[...][:, 0:1]
        h = s_ref[...] * dis + p1_ref[...] * dinv
        h = jnp.maximum(h, 0.0)
        p2 = jnp.dot(h, w_ref[...],
                     preferred_element_type=jnp.float32) + b_ref[...]
        p2_ref[...] = p2
        y_ref[...] = p2 * dis

    return pl.pallas_call(
        body,
        grid=(G,),
        in_specs=[
            pl.BlockSpec((BR, H), lambda i: (i, 0)),
            pl.BlockSpec((BR, H), lambda i: (i, 0)),
            pl.BlockSpec((BR, LANES), lambda i: (i, 0)),
            pl.BlockSpec((BR, LANES), lambda i: (i, 0)),
            pl.BlockSpec((H, C), lambda i: (0, 0)),
            pl.BlockSpec((1, C), lambda i: (0, 0)),
        ],
        out_specs=[
            pl.BlockSpec((BR, C), lambda i: (i, 0)),
            pl.BlockSpec((BR, C), lambda i: (i, 0)),
        ],
        out_shape=[
            jax.ShapeDtypeStruct((N, C), jnp.float32),
            jax.ShapeDtypeStruct((N, C), jnp.float32),
        ],
    )(S1, P1, disN, dinvN, W2, b2)


def _combine(S2, P2, disN, dinvN):
    N, C = S2.shape
    BR = 2000
    G = N // BR

    def body(s_ref, p_ref, dis_ref, dinv_ref, o_ref):
        o_ref[...] = (s_ref[...] * dis_ref[...][:, 0:1]
                      + p_ref[...] * dinv_ref[...][:, 0:1])

    return pl.pallas_call(
        body,
        grid=(G,),
        in_specs=[
            pl.BlockSpec((BR, C), lambda i: (i, 0)),
            pl.BlockSpec((BR, C), lambda i: (i, 0)),
            pl.BlockSpec((BR, LANES), lambda i: (i, 0)),
            pl.BlockSpec((BR, LANES), lambda i: (i, 0)),
        ],
        out_specs=pl.BlockSpec((BR, C), lambda i: (i, 0)),
        out_shape=jax.ShapeDtypeStruct((N, C), jnp.float32),
    )(S2, P2, disN, dinvN)


def kernel(X, he, W1, b1, W2, b2):
    N, D = X.shape
    M, K = he.shape
    C = W2.shape[1]

    Mp = _round_up(M, LANES * NW)
    Npad = _round_up(_round_up(N + 1, 128) // 128, NS) * 128
    E = 2 * M
    Ep = _round_up(E, NS * 128)

    he_i = he.astype(jnp.int32)
    he_pad = jnp.pad(he_i, ((0, Mp - M), (0, 0)))
    he_flat = he_pad.reshape(-1)
    he_cols = he_pad.T.reshape(-1)

    u_p, v_p, degw = _build_edges(X, he_flat, he_cols, M, Mp, K, Npad)
    u = u_p[:M]
    v = v_p[:M]
    src = jnp.pad(jnp.concatenate([u, v]), (0, Ep - E))
    dst = jnp.pad(jnp.concatenate([v, u]), (0, Ep - E), constant_values=N)
    lane = jnp.arange(LANES, dtype=jnp.int32)
    sidx_exp = (src[:, None] * LANES + lane).reshape(-1)
    didx_exp = (dst[:, None] * LANES + lane).reshape(-1)

    dis2, dinv2 = _deg_finalize(degw.reshape(NC, Npad // 128, 128), Npad)
    disN = jnp.broadcast_to(dis2.reshape(-1)[:N, None], (N, LANES))
    dinvN = jnp.broadcast_to(dinv2.reshape(-1)[:N, None], (N, LANES))

    Q1 = D // LANES
    Q2 = C // LANES
    P1, Y1 = _dense1(X, W1, b1.reshape(1, -1), disN)
    y1flat = Y1.reshape(N, Q1, LANES).transpose(1, 0, 2).reshape(-1)
    S1f = _scatter_pass(y1flat, sidx_exp, didx_exp, Q1, N, Npad)
    S1 = (S1f.reshape(Q1, Npad, LANES)[:, :N]
          .transpose(1, 0, 2).reshape(N, D))

    P2, Y2 = _dense2(S1, P1, disN, dinvN, W2, b2.reshape(1, -1))
    y2flat = Y2.reshape(N, Q2, LANES).transpose(1, 0, 2).reshape(-1)
    S2f = _scatter_pass(y2flat, sidx_exp, didx_exp, Q2, N, Npad)
    S2 = (S2f.reshape(Q2, Npad, LANES)[:, :N]
          .transpose(1, 0, 2).reshape(N, C))

    return _combine(S2, P2, disN, dinvN)
